# baseline (speedup 1.0000x reference)
import functools

import jax
import jax.numpy as jnp
from jax import lax
from jax.experimental import pallas as pl
from jax.experimental.pallas import tpu as pltpu
from jax.experimental.pallas import tpu_sc as plsc

D_MODEL = 64
X_ROWS = 16384
X_COLS = 50
N_TOKENS = X_ROWS * X_COLS

_INFO = plsc.get_sparse_core_info()
NUM_CORES = _INFO.num_cores
NUM_SUBCORES = _INFO.num_subcores
NW = NUM_CORES * NUM_SUBCORES
L = 128
XR_PER_W = X_ROWS // NW
ITB_PER_W = XR_PER_W // L
N_BLOCKS = ITB_PER_W * X_COLS
N_PAIRS = N_BLOCKS // 2
DT = D_MODEL // 8
SLAB_PAD = 129
UNROLL = 8
TBL_W = 128


def _transpose_block(rows_ref, slab_ref):
    k16 = lax.iota(jnp.int32, 16)
    dtds_idx = [k16 + dc * 16 for dc in range(D_MODEL // 16)]

    @plsc.parallel_loop(0, L, UNROLL, unroll=2)
    def il_body(i0):
        il_vec0 = jnp.full((16,), 0, jnp.int32) + i0
        for t in range(UNROLL):
            il = i0 + t
            il_vec = il_vec0 + t
            for dc in range(D_MODEL // 16):
                vec = rows_ref[il, pl.ds(dc * 16, 16)]
                plsc.store_scatter(slab_ref, [dtds_idx[dc], il_vec], vec)


def _emb_body(table_hbm, xt_hbm, out_hbm, xt_v, rows_v, slab_v, gsems, osems):
    wid = lax.axis_index("s") * NUM_CORES + lax.axis_index("c")
    i0 = wid * XR_PER_W
    it0 = wid * ITB_PER_W

    pltpu.sync_copy(xt_hbm.at[:, pl.ds(i0, XR_PER_W)], xt_v)

    def fire_gather(s, b):
        j = s // ITB_PER_W
        c = s % ITB_PER_W
        pltpu.async_copy(
            table_hbm.at[xt_v.at[j].at[pl.ds(c * L, L)]],
            rows_v.at[b],
            gsems.at[b],
        )

    def wait_gather(s, b):
        j = s // ITB_PER_W
        c = s % ITB_PER_W
        pltpu.make_async_copy(
            table_hbm.at[xt_v.at[j].at[pl.ds(c * L, L)]],
            rows_v.at[b],
            gsems.at[b],
        ).wait()

    def fire_out(s, b):
        j = s // ITB_PER_W
        c = s % ITB_PER_W
        for dt in range(DT):
            pltpu.async_copy(
                slab_v.at[b].at[pl.ds(dt * 8, 8), pl.ds(0, L)],
                out_hbm.at[j, dt, it0 + c],
                osems.at[b],
            )

    def wait_out(s, b):
        j = s // ITB_PER_W
        c = s % ITB_PER_W
        for dt in range(DT):
            pltpu.make_async_copy(
                slab_v.at[b].at[pl.ds(dt * 8, 8), pl.ds(0, L)],
                out_hbm.at[j, dt, it0 + c],
                osems.at[b],
            ).wait()

    fire_gather(0, 0)

    def outer(g, carry):
        s0 = 2 * g
        wait_gather(s0, 0)
        fire_gather(s0 + 1, 1)
        pl.when(g > 0)(lambda: wait_out(s0 - 2, 0))
        _transpose_block(rows_v.at[0], slab_v.at[0])
        fire_out(s0, 0)
        wait_gather(s0 + 1, 1)
        pl.when(g > 0)(lambda: wait_out(s0 - 1, 1))
        _transpose_block(rows_v.at[1], slab_v.at[1])
        fire_out(s0 + 1, 1)
        pl.when(g < N_PAIRS - 1)(lambda: fire_gather(s0 + 2, 0))
        return carry

    lax.fori_loop(0, N_PAIRS, outer, 0)

    wait_out(N_BLOCKS - 2, 0)
    wait_out(N_BLOCKS - 1, 1)


@functools.partial(
    pl.kernel,
    out_type=jax.ShapeDtypeStruct((X_COLS, DT, X_ROWS // L, 8, L), jnp.float32),
    mesh=plsc.VectorSubcoreMesh(core_axis_name="c", subcore_axis_name="s"),
    compiler_params=pltpu.CompilerParams(
        use_tc_tiling_on_sc=False, needs_layout_passes=False
    ),
    scratch_types=[
        pltpu.VMEM((X_COLS, XR_PER_W), jnp.int32),
        pltpu.VMEM((2, L, TBL_W), jnp.float32),
        pltpu.VMEM((2, D_MODEL, SLAB_PAD), jnp.float32),
        pltpu.SemaphoreType.DMA((2,)),
        pltpu.SemaphoreType.DMA((2,)),
    ],
)
def _emb_kernel(table_hbm, xt_hbm, out_hbm, xt_v, rows_v, slab_v, gsems, osems):
    _emb_body(table_hbm, xt_hbm, out_hbm, xt_v, rows_v, slab_v, gsems, osems)


def kernel(x, table):
    t128 = jnp.pad(table, ((0, 0), (0, TBL_W - D_MODEL)))
    out5 = _emb_kernel(t128, x.T.astype(jnp.int32))
    return out5.transpose(2, 4, 0, 1, 3).reshape(X_ROWS, X_COLS, D_MODEL)


# speedup vs baseline: 1.0216x; 1.0216x over previous
import functools

import jax
import jax.numpy as jnp
from jax import lax
from jax.experimental import pallas as pl
from jax.experimental.pallas import tpu as pltpu
from jax.experimental.pallas import tpu_sc as plsc

D_MODEL = 64
X_ROWS = 16384
X_COLS = 50
N_TOKENS = X_ROWS * X_COLS

_INFO = plsc.get_sparse_core_info()
NUM_CORES = _INFO.num_cores
NUM_SUBCORES = _INFO.num_subcores
NW = NUM_CORES * NUM_SUBCORES
L = 128
XR_PER_W = X_ROWS // NW
ITB_PER_W = XR_PER_W // L
N_BLOCKS = ITB_PER_W * X_COLS
N_PAIRS = N_BLOCKS // 2
DT = D_MODEL // 8
SLAB_PAD = 129
UNROLL = 4
TBL_W = 128


def _transpose_block(rows_ref, slab_ref):
    k16 = lax.iota(jnp.int32, 16)
    dtds_idx = [k16 + dc * 16 for dc in range(D_MODEL // 16)]

    @plsc.parallel_loop(0, L, UNROLL)
    def il_body(i0):
        il_vec0 = jnp.full((16,), 0, jnp.int32) + i0
        for t in range(UNROLL):
            il = i0 + t
            il_vec = il_vec0 + t
            for dc in range(D_MODEL // 16):
                vec = rows_ref[il, pl.ds(dc * 16, 16)]
                plsc.store_scatter(slab_ref, [dtds_idx[dc], il_vec], vec)


def _emb_body(table_hbm, xt_hbm, out_hbm, xt_v, rows_v, slab_v, gsems, osems):
    wid = lax.axis_index("s") * NUM_CORES + lax.axis_index("c")
    i0 = wid * XR_PER_W
    it0 = wid * ITB_PER_W

    pltpu.sync_copy(xt_hbm.at[:, pl.ds(i0, XR_PER_W)], xt_v)

    def fire_gather(s, b):
        j = s // ITB_PER_W
        c = s % ITB_PER_W
        pltpu.async_copy(
            table_hbm.at[xt_v.at[j].at[pl.ds(c * L, L)]],
            rows_v.at[b],
            gsems.at[b],
        )

    def wait_gather(s, b):
        j = s // ITB_PER_W
        c = s % ITB_PER_W
        pltpu.make_async_copy(
            table_hbm.at[xt_v.at[j].at[pl.ds(c * L, L)]],
            rows_v.at[b],
            gsems.at[b],
        ).wait()

    def fire_out(s, b):
        j = s // ITB_PER_W
        c = s % ITB_PER_W
        for dt in range(DT):
            pltpu.async_copy(
                slab_v.at[b].at[pl.ds(dt * 8, 8), pl.ds(0, L)],
                out_hbm.at[j, dt, it0 + c],
                osems.at[b],
            )

    def wait_out(s, b):
        j = s // ITB_PER_W
        c = s % ITB_PER_W
        for dt in range(DT):
            pltpu.make_async_copy(
                slab_v.at[b].at[pl.ds(dt * 8, 8), pl.ds(0, L)],
                out_hbm.at[j, dt, it0 + c],
                osems.at[b],
            ).wait()

    fire_gather(0, 0)

    def outer(g, carry):
        s0 = 2 * g
        wait_gather(s0, 0)
        fire_gather(s0 + 1, 1)
        pl.when(g > 0)(lambda: wait_out(s0 - 2, 0))
        _transpose_block(rows_v.at[0], slab_v.at[0])
        fire_out(s0, 0)
        wait_gather(s0 + 1, 1)
        pl.when(g > 0)(lambda: wait_out(s0 - 1, 1))
        _transpose_block(rows_v.at[1], slab_v.at[1])
        fire_out(s0 + 1, 1)
        pl.when(g < N_PAIRS - 1)(lambda: fire_gather(s0 + 2, 0))
        return carry

    lax.fori_loop(0, N_PAIRS, outer, 0)

    wait_out(N_BLOCKS - 2, 0)
    wait_out(N_BLOCKS - 1, 1)


@functools.partial(
    pl.kernel,
    out_type=jax.ShapeDtypeStruct((X_COLS, DT, X_ROWS // L, 8, L), jnp.float32),
    mesh=plsc.VectorSubcoreMesh(core_axis_name="c", subcore_axis_name="s"),
    compiler_params=pltpu.CompilerParams(
        use_tc_tiling_on_sc=False, needs_layout_passes=False
    ),
    scratch_types=[
        pltpu.VMEM((X_COLS, XR_PER_W), jnp.int32),
        pltpu.VMEM((2, L, TBL_W), jnp.float32),
        pltpu.VMEM((2, D_MODEL, SLAB_PAD), jnp.float32),
        pltpu.SemaphoreType.DMA((2,)),
        pltpu.SemaphoreType.DMA((2,)),
    ],
)
def _emb_kernel(table_hbm, xt_hbm, out_hbm, xt_v, rows_v, slab_v, gsems, osems):
    _emb_body(table_hbm, xt_hbm, out_hbm, xt_v, rows_v, slab_v, gsems, osems)


def kernel(x, table):
    t128 = jnp.pad(table, ((0, 0), (0, TBL_W - D_MODEL)))
    out5 = _emb_kernel(t128, x.T.astype(jnp.int32))
    return out5.transpose(2, 4, 0, 1, 3).reshape(X_ROWS, X_COLS, D_MODEL)
